# baseline (speedup 1.0000x reference)
import functools

import jax
import jax.numpy as jnp
from jax import lax
from jax.experimental import pallas as pl
from jax.experimental.pallas import tpu as pltpu
from jax.experimental.pallas import tpu_sc as plsc

_NC = 2
_NS = 16
_CHUNK = 128
_BN_EPS = 1e-5

_MESH = plsc.VectorSubcoreMesh(core_axis_name="c", subcore_axis_name="s",
                               num_cores=_NC, num_subcores=_NS)
_CP = pltpu.CompilerParams(needs_layout_passes=False)


def _cdiv(a, b):
    return (a + b - 1) // b


def _make_sc_deg(NP, NCH):
    ZT = NP // _NS

    @functools.partial(
        pl.kernel, out_type=jax.ShapeDtypeStruct((_NC * NP,), jnp.float32),
        mesh=_MESH, compiler_params=_CP,
        scratch_types=[
            pltpu.VMEM_SHARED((NP,), jnp.float32),
            pltpu.VMEM((8, _CHUNK), jnp.int32),
            pltpu.VMEM((_CHUNK,), jnp.float32),
            pltpu.SemaphoreType.DMA,
        ])
    def sc_deg(coli_hbm, z1_hbm, ones_hbm, deg_hbm, deg_s, cidx8, ones_v,
               ssem):
        c = lax.axis_index("c")
        s = lax.axis_index("s")
        r_lo = s * ZT
        pltpu.sync_copy(ones_hbm, ones_v)
        pltpu.sync_copy(z1_hbm, deg_s.at[pl.ds(r_lo, ZT)])
        plsc.subcore_barrier()

        def deg_body(jb, _):
            pltpu.sync_copy(coli_hbm.at[c, s, pl.ds(jb * 8, 8)], cidx8)
            for jj in range(8):
                pltpu.async_copy(ones_v, deg_s.at[cidx8.at[jj]], ssem,
                                 add=True).wait()
            return 0
        lax.fori_loop(0, NCH // 8, deg_body, 0)
        plsc.subcore_barrier()

        pltpu.sync_copy(deg_s.at[pl.ds(r_lo, ZT)],
                        deg_hbm.at[pl.ds(c * NP + r_lo, ZT)])

    return sc_deg


def _make_sc_hop(NP, D, NCH):
    ZT = NP // _NS
    ZC = 160
    assert ZT % ZC == 0

    @functools.partial(
        pl.kernel,
        out_type=jax.ShapeDtypeStruct((_NC, NP, D), jnp.float32),
        mesh=_MESH, compiler_params=_CP,
        scratch_types=[
            pltpu.VMEM_SHARED((NP, D), jnp.float32),
            pltpu.VMEM((2, 8, _CHUNK), jnp.int32),
            pltpu.VMEM((2, 8, _CHUNK), jnp.int32),
            pltpu.VMEM((2, _CHUNK, D), jnp.float32),
            pltpu.SemaphoreType.DMA,
            pltpu.SemaphoreType.DMA,
            pltpu.SemaphoreType.DMA,
        ])
    def sc_hop(u_hbm, rowi_hbm, coli_hbm, z2_hbm, t_hbm,
               acc_s, ridx, cidx, gbuf, gsem, ssem, isem):
        c = lax.axis_index("c")
        s = lax.axis_index("s")
        r_lo = s * ZT
        for q in range(ZT // ZC):
            pltpu.sync_copy(z2_hbm, acc_s.at[pl.ds(r_lo + q * ZC, ZC)])
        plsc.subcore_barrier()

        NB = NCH // 8
        T = NB * 8

        def r_at(q):
            return ridx.at[(q // 8) % 2, q % 8]

        def c_at(q):
            return cidx.at[(q // 8) % 2, q % 8]

        pltpu.sync_copy(rowi_hbm.at[c, s, pl.ds(0, 8)], ridx.at[0])
        pltpu.sync_copy(coli_hbm.at[c, s, pl.ds(0, 8)], cidx.at[0])
        g = pltpu.async_copy(u_hbm.at[r_at(0)], gbuf.at[0], gsem)
        ifr = ifc = None
        scats = [None, None]
        for q in range(T):
            jb = q // 8
            if q % 8 == 0 and jb + 1 < NB:
                ifr = pltpu.async_copy(
                    rowi_hbm.at[c, s, pl.ds((jb + 1) * 8, 8)],
                    ridx.at[(jb + 1) % 2], isem)
                ifc = pltpu.async_copy(
                    coli_hbm.at[c, s, pl.ds((jb + 1) * 8, 8)],
                    cidx.at[(jb + 1) % 2], isem)
            g.wait()
            sd = pltpu.async_copy(gbuf.at[q % 2], acc_s.at[c_at(q)], ssem,
                                  add=True)
            if q + 1 < T:
                if (q + 1) % 8 == 0:
                    ifr.wait()
                    ifc.wait()
                if scats[(q + 1) % 2] is not None:
                    scats[(q + 1) % 2].wait()
                    scats[(q + 1) % 2] = None
                g = pltpu.async_copy(u_hbm.at[r_at(q + 1)],
                                     gbuf.at[(q + 1) % 2], gsem)
            scats[q % 2] = sd
        for sd in scats:
            if sd is not None:
                sd.wait()
        plsc.subcore_barrier()

        pltpu.sync_copy(acc_s.at[pl.ds(r_lo, ZT)],
                        t_hbm.at[c, pl.ds(r_lo, ZT)])

    return sc_hop


def _u0_body(x_ref, dega_ref, degb_ref, u_ref):
    deg = dega_ref[...] + degb_ref[...]
    dis = jnp.where(deg > 0.5, lax.rsqrt(deg), 0.0)
    u_ref[...] = x_ref[...] * dis


def _uh_body(N):
    def body(t_ref, dega_ref, degb_ref, u_ref):
        deg = dega_ref[...] + degb_ref[...]
        dis2 = jnp.where(deg > 0.5, 1.0 / deg, 0.0)
        u_ref[...] = (t_ref[0, :N] + t_ref[1, :N]) * dis2
    return body


def _branch_body(N, is_hop, has_acc):
    def body(*refs):
        it = iter(refs)
        xin_ref = next(it)
        dega_ref = next(it)
        degb_ref = next(it)
        Wk_ref, bk_ref, gk_ref, bek_ref, Wok_ref = (
            next(it), next(it), next(it), next(it), next(it))
        bo_ref = None if has_acc else next(it)
        acc_ref = next(it) if has_acc else None
        o_ref = next(it)

        if is_hop:
            deg = dega_ref[...] + degb_ref[...]
            dis = jnp.where(deg > 0.5, lax.rsqrt(deg), 0.0)
            xk = (xin_ref[0, :N] + xin_ref[1, :N]) * dis
        else:
            xk = xin_ref[...]
        h = lax.dot_general(
            xk, Wk_ref[...], (((1,), (1,)), ((), ())),
            preferred_element_type=jnp.float32,
            precision=lax.Precision.DEFAULT) + bk_ref[...][None, :]
        mean = jnp.mean(h, axis=0, keepdims=True)
        var = jnp.mean(jnp.square(h - mean), axis=0, keepdims=True)
        h = (h - mean) * lax.rsqrt(var + _BN_EPS) * gk_ref[...][None, :]
        h = h + bek_ref[...][None, :]
        h = jnp.maximum(h, 0.0)
        p = lax.dot_general(
            h, Wok_ref[...], (((1,), (1,)), ((), ())),
            preferred_element_type=jnp.float32,
            precision=lax.Precision.DEFAULT)
        if has_acc:
            o_ref[...] = acc_ref[...] + p
        else:
            o_ref[...] = p + bo_ref[...][None, :]

    return body


def kernel(x, W, b, gamma, beta, W_out, b_out, edge_index, num_nodes):
    N, D = x.shape
    K = W.shape[0] - 1
    E = edge_index.shape[1]
    NP = _cdiv(N + 16, 512) * 512
    NW = _NC * _NS

    row = edge_index[0]
    col = edge_index[1]

    assert E % NW == 0
    ET = E // NW
    ETP = _cdiv(ET, _CHUNK * 8) * (_CHUNK * 8)
    NCH = ETP // _CHUNK
    npad = ETP - ET
    lanes = (jnp.arange(npad, dtype=jnp.int32) % 16)
    row_t = jnp.concatenate(
        [row.reshape(NW, ET),
         jnp.broadcast_to(lanes[None, :], (NW, npad))], axis=1)
    col_t = jnp.concatenate(
        [col.reshape(NW, ET),
         jnp.broadcast_to(N + lanes[None, :], (NW, npad))], axis=1)
    row_t = row_t.reshape(_NC, _NS, NCH, _CHUNK)
    col_t = col_t.reshape(_NC, _NS, NCH, _CHUNK)

    ZT = NP // _NS
    z1 = jnp.zeros((ZT,), jnp.float32)
    z2 = jnp.zeros((160, D), jnp.float32)
    ones = jnp.ones((_CHUNK,), jnp.float32)

    sc_deg = _make_sc_deg(NP, NCH)
    deg2 = sc_deg(col_t, z1, ones)
    deg_a = deg2[:N, None]
    deg_b = deg2[NP:NP + N, None]

    sc_hop = _make_sc_hop(NP, D, NCH)
    D_OUT = W_out.shape[0]
    D_HID = W.shape[1]

    u = pl.pallas_call(
        _u0_body, out_shape=jax.ShapeDtypeStruct((N, D), jnp.float32),
    )(x, deg_a, deg_b)

    t_list = []
    for h in range(K):
        t = sc_hop(u, row_t, col_t, z2)
        t_list.append(t)
        if h + 1 < K:
            u = pl.pallas_call(
                _uh_body(N),
                out_shape=jax.ShapeDtypeStruct((N, D), jnp.float32),
            )(t, deg_a, deg_b)

    acc = None
    for k in range(K + 1):
        xin = x if k == 0 else t_list[k - 1]
        args = [xin, deg_a, deg_b, W[k], b[k], gamma[k], beta[k],
                W_out[:, k * D_HID:(k + 1) * D_HID]]
        args.append(b_out if acc is None else acc)
        acc = pl.pallas_call(
            _branch_body(N, k > 0, acc is not None),
            out_shape=jax.ShapeDtypeStruct((N, D_OUT), jnp.float32),
        )(*args)
    return acc


# speedup vs baseline: 1.0004x; 1.0004x over previous
import functools

import jax
import jax.numpy as jnp
from jax import lax
from jax.experimental import pallas as pl
from jax.experimental.pallas import tpu as pltpu
from jax.experimental.pallas import tpu_sc as plsc

_NC = 2
_NS = 16
_CHUNK = 128
_BN_EPS = 1e-5

_MESH = plsc.VectorSubcoreMesh(core_axis_name="c", subcore_axis_name="s",
                               num_cores=_NC, num_subcores=_NS)
_CP = pltpu.CompilerParams(needs_layout_passes=False)


def _cdiv(a, b):
    return (a + b - 1) // b


def _make_sc_deg(NP, NCH):
    ZT = NP // _NS

    @functools.partial(
        pl.kernel, out_type=jax.ShapeDtypeStruct((_NC * NP,), jnp.float32),
        mesh=_MESH, compiler_params=_CP,
        scratch_types=[
            pltpu.VMEM_SHARED((NP,), jnp.float32),
            pltpu.VMEM((8, _CHUNK), jnp.int32),
            pltpu.VMEM((_CHUNK,), jnp.float32),
            pltpu.SemaphoreType.DMA,
        ])
    def sc_deg(coli_hbm, z1_hbm, ones_hbm, deg_hbm, deg_s, cidx8, ones_v,
               ssem):
        c = lax.axis_index("c")
        s = lax.axis_index("s")
        r_lo = s * ZT
        pltpu.sync_copy(ones_hbm, ones_v)
        pltpu.sync_copy(z1_hbm, deg_s.at[pl.ds(r_lo, ZT)])
        plsc.subcore_barrier()

        def deg_body(jb, _):
            pltpu.sync_copy(coli_hbm.at[c, s, pl.ds(jb * 8, 8)], cidx8)
            for jj in range(8):
                pltpu.async_copy(ones_v, deg_s.at[cidx8.at[jj]], ssem,
                                 add=True).wait()
            return 0
        lax.fori_loop(0, NCH // 8, deg_body, 0)
        plsc.subcore_barrier()

        pltpu.sync_copy(deg_s.at[pl.ds(r_lo, ZT)],
                        deg_hbm.at[pl.ds(c * NP + r_lo, ZT)])

    return sc_deg


def _make_sc_hop(NP, D, NCH):
    ZT = NP // _NS
    ZC = 160
    assert ZT % ZC == 0

    @functools.partial(
        pl.kernel,
        out_type=jax.ShapeDtypeStruct((_NC, NP, D), jnp.float32),
        mesh=_MESH, compiler_params=_CP,
        scratch_types=[
            pltpu.VMEM_SHARED((NP, D), jnp.float32),
            pltpu.VMEM((2, 8, _CHUNK), jnp.int32),
            pltpu.VMEM((2, 8, _CHUNK), jnp.int32),
            pltpu.VMEM((2, _CHUNK, D), jnp.float32),
            pltpu.SemaphoreType.DMA,
            pltpu.SemaphoreType.DMA,
            pltpu.SemaphoreType.DMA,
        ])
    def sc_hop(u_hbm, rowi_hbm, coli_hbm, z2_hbm, t_hbm,
               acc_s, ridx, cidx, gbuf, gsem, ssem, isem):
        c = lax.axis_index("c")
        s = lax.axis_index("s")
        r_lo = s * ZT
        for q in range(ZT // ZC):
            pltpu.sync_copy(z2_hbm, acc_s.at[pl.ds(r_lo + q * ZC, ZC)])
        plsc.subcore_barrier()

        NB = NCH // 8
        T = NB * 8

        def r_at(q):
            return ridx.at[(q // 8) % 2, q % 8]

        def c_at(q):
            return cidx.at[(q // 8) % 2, q % 8]

        pltpu.sync_copy(rowi_hbm.at[c, s, pl.ds(0, 8)], ridx.at[0])
        pltpu.sync_copy(coli_hbm.at[c, s, pl.ds(0, 8)], cidx.at[0])
        g = pltpu.async_copy(u_hbm.at[r_at(0)], gbuf.at[0], gsem)
        ifr = ifc = None
        scats = [None, None]
        for q in range(T):
            jb = q // 8
            if q % 8 == 0 and jb + 1 < NB:
                ifr = pltpu.async_copy(
                    rowi_hbm.at[c, s, pl.ds((jb + 1) * 8, 8)],
                    ridx.at[(jb + 1) % 2], isem)
                ifc = pltpu.async_copy(
                    coli_hbm.at[c, s, pl.ds((jb + 1) * 8, 8)],
                    cidx.at[(jb + 1) % 2], isem)
            g.wait()
            sd = pltpu.async_copy(gbuf.at[q % 2], acc_s.at[c_at(q)], ssem,
                                  add=True)
            if q + 1 < T:
                if (q + 1) % 8 == 0:
                    ifr.wait()
                    ifc.wait()
                if scats[(q + 1) % 2] is not None:
                    scats[(q + 1) % 2].wait()
                    scats[(q + 1) % 2] = None
                g = pltpu.async_copy(u_hbm.at[r_at(q + 1)],
                                     gbuf.at[(q + 1) % 2], gsem)
            scats[q % 2] = sd
        for sd in scats:
            if sd is not None:
                sd.wait()
        plsc.subcore_barrier()

        pltpu.sync_copy(acc_s.at[pl.ds(r_lo, ZT)],
                        t_hbm.at[c, pl.ds(r_lo, ZT)])

    return sc_hop


def _branch_body(N, is_hop, has_acc, emit_u):
    def body(*refs):
        it = iter(refs)
        xin_ref = next(it)
        dega_ref = next(it)
        degb_ref = next(it)
        Wk_ref, bk_ref, gk_ref, bek_ref, Wok_ref = (
            next(it), next(it), next(it), next(it), next(it))
        bo_ref = None if has_acc else next(it)
        acc_ref = next(it) if has_acc else None
        o_ref = next(it)
        u_ref = next(it) if emit_u else None

        deg = dega_ref[...] + degb_ref[...]
        dis = jnp.where(deg > 0.5, lax.rsqrt(deg), 0.0)
        if is_hop:
            xk = (xin_ref[0, :N] + xin_ref[1, :N]) * dis
        else:
            xk = xin_ref[...]
        if emit_u:
            u_ref[...] = xk * dis
        h = lax.dot_general(
            xk, Wk_ref[...], (((1,), (1,)), ((), ())),
            preferred_element_type=jnp.float32,
            precision=lax.Precision.DEFAULT) + bk_ref[...][None, :]
        mean = jnp.mean(h, axis=0, keepdims=True)
        var = jnp.mean(jnp.square(h - mean), axis=0, keepdims=True)
        h = (h - mean) * lax.rsqrt(var + _BN_EPS) * gk_ref[...][None, :]
        h = h + bek_ref[...][None, :]
        h = jnp.maximum(h, 0.0)
        p = lax.dot_general(
            h, Wok_ref[...], (((1,), (1,)), ((), ())),
            preferred_element_type=jnp.float32,
            precision=lax.Precision.DEFAULT)
        if has_acc:
            o_ref[...] = acc_ref[...] + p
        else:
            o_ref[...] = p + bo_ref[...][None, :]

    return body


def kernel(x, W, b, gamma, beta, W_out, b_out, edge_index, num_nodes):
    N, D = x.shape
    K = W.shape[0] - 1
    E = edge_index.shape[1]
    NP = _cdiv(N + 16, 512) * 512
    NW = _NC * _NS

    row = edge_index[0]
    col = edge_index[1]

    assert E % NW == 0
    ET = E // NW
    ETP = _cdiv(ET, _CHUNK * 8) * (_CHUNK * 8)
    NCH = ETP // _CHUNK
    npad = ETP - ET
    lanes = (jnp.arange(npad, dtype=jnp.int32) % 16)
    row_t = jnp.concatenate(
        [row.reshape(NW, ET),
         jnp.broadcast_to(lanes[None, :], (NW, npad))], axis=1)
    col_t = jnp.concatenate(
        [col.reshape(NW, ET),
         jnp.broadcast_to(N + lanes[None, :], (NW, npad))], axis=1)
    row_t = row_t.reshape(_NC, _NS, NCH, _CHUNK)
    col_t = col_t.reshape(_NC, _NS, NCH, _CHUNK)

    ZT = NP // _NS
    z1 = jnp.zeros((ZT,), jnp.float32)
    z2 = jnp.zeros((160, D), jnp.float32)
    ones = jnp.ones((_CHUNK,), jnp.float32)

    sc_deg = _make_sc_deg(NP, NCH)
    deg2 = sc_deg(col_t, z1, ones)
    deg_a = deg2[:N, None]
    deg_b = deg2[NP:NP + N, None]

    sc_hop = _make_sc_hop(NP, D, NCH)
    D_OUT = W_out.shape[0]
    D_HID = W.shape[1]
    acc = None
    xin = x
    for k in range(K + 1):
        emit_u = k < K
        out_shape = [jax.ShapeDtypeStruct((N, D_OUT), jnp.float32)]
        if emit_u:
            out_shape.append(jax.ShapeDtypeStruct((N, D), jnp.float32))
        args = [xin, deg_a, deg_b, W[k], b[k], gamma[k], beta[k],
                W_out[:, k * D_HID:(k + 1) * D_HID]]
        args.append(b_out if acc is None else acc)
        res = pl.pallas_call(
            _branch_body(N, k > 0, acc is not None, emit_u),
            out_shape=out_shape,
        )(*args)
        acc = res[0]
        if emit_u:
            xin = sc_hop(res[1], row_t, col_t, z2)
    return acc


# speedup vs baseline: 1.0132x; 1.0128x over previous
import functools

import jax
import jax.numpy as jnp
from jax import lax
from jax.experimental import pallas as pl
from jax.experimental.pallas import tpu as pltpu
from jax.experimental.pallas import tpu_sc as plsc

_NC = 2
_NS = 16
_CHUNK = 128
_BN_EPS = 1e-5

_MESH = plsc.VectorSubcoreMesh(core_axis_name="c", subcore_axis_name="s",
                               num_cores=_NC, num_subcores=_NS)
_CP = pltpu.CompilerParams(needs_layout_passes=False)


def _cdiv(a, b):
    return (a + b - 1) // b


def _make_sc_deg(NP, NCH):
    ZT = NP // _NS

    @functools.partial(
        pl.kernel, out_type=jax.ShapeDtypeStruct((NP,), jnp.float32),
        mesh=_MESH, compiler_params=_CP,
        scratch_types=[
            pltpu.VMEM_SHARED((NP,), jnp.float32),
            pltpu.VMEM((8, _CHUNK), jnp.int32),
            pltpu.VMEM((_CHUNK,), jnp.float32),
            pltpu.SemaphoreType.DMA,
        ])
    def sc_deg(coli_hbm, z1_hbm, ones_hbm, deg_hbm, deg_s, cidx8, ones_v,
               ssem):
        c = lax.axis_index("c")
        s = lax.axis_index("s")
        r_lo = s * ZT
        pltpu.sync_copy(ones_hbm, ones_v)
        pltpu.sync_copy(z1_hbm, deg_s.at[pl.ds(r_lo, ZT)])
        plsc.subcore_barrier()
        for half in range(_NC):
            def deg_body(jb, _):
                pltpu.sync_copy(coli_hbm.at[half, s, pl.ds(jb * 8, 8)],
                                cidx8)
                for jj in range(8):
                    pltpu.async_copy(ones_v, deg_s.at[cidx8.at[jj]], ssem,
                                     add=True).wait()
                return 0
            lax.fori_loop(0, NCH // 8, deg_body, 0)
        plsc.subcore_barrier()

        @pl.when(c == 0)
        def _():
            pltpu.sync_copy(deg_s.at[pl.ds(r_lo, ZT)],
                            deg_hbm.at[pl.ds(r_lo, ZT)])

    return sc_deg


def _make_sc_hop(NP, D, NCH):
    ZT = NP // _NS
    ZC = 160
    assert ZT % ZC == 0

    @functools.partial(
        pl.kernel,
        out_type=jax.ShapeDtypeStruct((_NC, NP, D), jnp.float32),
        mesh=_MESH, compiler_params=_CP,
        scratch_types=[
            pltpu.VMEM_SHARED((NP, D), jnp.float32),
            pltpu.VMEM((2, 8, _CHUNK), jnp.int32),
            pltpu.VMEM((2, 8, _CHUNK), jnp.int32),
            pltpu.VMEM((2, _CHUNK, D), jnp.float32),
            pltpu.SemaphoreType.DMA,
            pltpu.SemaphoreType.DMA,
            pltpu.SemaphoreType.DMA,
        ])
    def sc_hop(u_hbm, rowi_hbm, coli_hbm, z2_hbm, t_hbm,
               acc_s, ridx, cidx, gbuf, gsem, ssem, isem):
        c = lax.axis_index("c")
        s = lax.axis_index("s")
        r_lo = s * ZT
        for q in range(ZT // ZC):
            pltpu.sync_copy(z2_hbm, acc_s.at[pl.ds(r_lo + q * ZC, ZC)])
        plsc.subcore_barrier()

        NB = NCH // 8
        T = NB * 8

        def r_at(q):
            return ridx.at[(q // 8) % 2, q % 8]

        def c_at(q):
            return cidx.at[(q // 8) % 2, q % 8]

        pltpu.sync_copy(rowi_hbm.at[c, s, pl.ds(0, 8)], ridx.at[0])
        pltpu.sync_copy(coli_hbm.at[c, s, pl.ds(0, 8)], cidx.at[0])
        g = pltpu.async_copy(u_hbm.at[r_at(0)], gbuf.at[0], gsem)
        ifr = ifc = None
        scats = [None, None]
        for q in range(T):
            jb = q // 8
            if q % 8 == 0 and jb + 1 < NB:
                ifr = pltpu.async_copy(
                    rowi_hbm.at[c, s, pl.ds((jb + 1) * 8, 8)],
                    ridx.at[(jb + 1) % 2], isem)
                ifc = pltpu.async_copy(
                    coli_hbm.at[c, s, pl.ds((jb + 1) * 8, 8)],
                    cidx.at[(jb + 1) % 2], isem)
            g.wait()
            sd = pltpu.async_copy(gbuf.at[q % 2], acc_s.at[c_at(q)], ssem,
                                  add=True)
            if q + 1 < T:
                if (q + 1) % 8 == 0:
                    ifr.wait()
                    ifc.wait()
                if scats[(q + 1) % 2] is not None:
                    scats[(q + 1) % 2].wait()
                    scats[(q + 1) % 2] = None
                g = pltpu.async_copy(u_hbm.at[r_at(q + 1)],
                                     gbuf.at[(q + 1) % 2], gsem)
            scats[q % 2] = sd
        for sd in scats:
            if sd is not None:
                sd.wait()
        plsc.subcore_barrier()

        pltpu.sync_copy(acc_s.at[pl.ds(r_lo, ZT)],
                        t_hbm.at[c, pl.ds(r_lo, ZT)])

    return sc_hop


def _branch_body(N, is_hop, has_acc, emit_u):
    def body(*refs):
        it = iter(refs)
        xin_ref = next(it)
        deg_ref = next(it)
        Wk_ref, bk_ref, gk_ref, bek_ref, Wok_ref = (
            next(it), next(it), next(it), next(it), next(it))
        bo_ref = None if has_acc else next(it)
        acc_ref = next(it) if has_acc else None
        o_ref = next(it)
        u_ref = next(it) if emit_u else None

        deg = deg_ref[...]
        dis = jnp.where(deg > 0.5, lax.rsqrt(deg), 0.0)
        if is_hop:
            xk = (xin_ref[0, :N] + xin_ref[1, :N]) * dis
        else:
            xk = xin_ref[...]
        if emit_u:
            u_ref[...] = xk * dis
        h = lax.dot_general(
            xk, Wk_ref[...], (((1,), (1,)), ((), ())),
            preferred_element_type=jnp.float32,
            precision=lax.Precision.DEFAULT) + bk_ref[...][None, :]
        mean = jnp.mean(h, axis=0, keepdims=True)
        var = jnp.mean(jnp.square(h - mean), axis=0, keepdims=True)
        h = (h - mean) * lax.rsqrt(var + _BN_EPS) * gk_ref[...][None, :]
        h = h + bek_ref[...][None, :]
        h = jnp.maximum(h, 0.0)
        p = lax.dot_general(
            h, Wok_ref[...], (((1,), (1,)), ((), ())),
            preferred_element_type=jnp.float32,
            precision=lax.Precision.DEFAULT)
        if has_acc:
            o_ref[...] = acc_ref[...] + p
        else:
            o_ref[...] = p + bo_ref[...][None, :]

    return body


def kernel(x, W, b, gamma, beta, W_out, b_out, edge_index, num_nodes):
    N, D = x.shape
    K = W.shape[0] - 1
    E = edge_index.shape[1]
    NP = _cdiv(N + 16, 512) * 512
    NW = _NC * _NS

    row = edge_index[0]
    col = edge_index[1]

    assert E % NW == 0
    ET = E // NW
    ETP = _cdiv(ET, _CHUNK * 8) * (_CHUNK * 8)
    NCH = ETP // _CHUNK
    npad = ETP - ET
    lanes = (jnp.arange(npad, dtype=jnp.int32) % 16)
    row_t = jnp.concatenate(
        [row.reshape(NW, ET),
         jnp.broadcast_to(lanes[None, :], (NW, npad))], axis=1)
    col_t = jnp.concatenate(
        [col.reshape(NW, ET),
         jnp.broadcast_to(N + lanes[None, :], (NW, npad))], axis=1)
    row_t = row_t.reshape(_NC, _NS, NCH, _CHUNK)
    col_t = col_t.reshape(_NC, _NS, NCH, _CHUNK)

    ZT = NP // _NS
    z1 = jnp.zeros((ZT,), jnp.float32)
    z2 = jnp.zeros((160, D), jnp.float32)
    ones = jnp.ones((_CHUNK,), jnp.float32)

    sc_deg = _make_sc_deg(NP, NCH)
    deg = sc_deg(col_t, z1, ones)
    deg_n = deg[:N, None]

    sc_hop = _make_sc_hop(NP, D, NCH)
    D_OUT = W_out.shape[0]
    D_HID = W.shape[1]
    acc = None
    xin = x
    for k in range(K + 1):
        emit_u = k < K
        out_shape = [jax.ShapeDtypeStruct((N, D_OUT), jnp.float32)]
        if emit_u:
            out_shape.append(jax.ShapeDtypeStruct((N, D), jnp.float32))
        args = [xin, deg_n, W[k], b[k], gamma[k], beta[k],
                W_out[:, k * D_HID:(k + 1) * D_HID]]
        args.append(b_out if acc is None else acc)
        res = pl.pallas_call(
            _branch_body(N, k > 0, acc is not None, emit_u),
            out_shape=out_shape,
        )(*args)
        acc = res[0]
        if emit_u:
            xin = sc_hop(res[1], row_t, col_t, z2)
    return acc


# speedup vs baseline: 1.0142x; 1.0011x over previous
import functools

import jax
import jax.numpy as jnp
from jax import lax
from jax.experimental import pallas as pl
from jax.experimental.pallas import tpu as pltpu
from jax.experimental.pallas import tpu_sc as plsc

_NC = 2
_NS = 16
_CHUNK = 128
_BN_EPS = 1e-5

_CP = pltpu.CompilerParams(needs_layout_passes=False)


def _mesh():
    return plsc.VectorSubcoreMesh(core_axis_name="c", subcore_axis_name="s",
                                  num_cores=_NC, num_subcores=_NS)


def _cdiv(a, b):
    return (a + b - 1) // b


def _make_sc_deg(NP, NCH):
    ZT = NP // _NS

    @functools.partial(
        pl.kernel, out_type=jax.ShapeDtypeStruct((NP,), jnp.float32),
        mesh=_mesh(), compiler_params=_CP,
        scratch_types=[
            pltpu.VMEM_SHARED((NP,), jnp.float32),
            pltpu.VMEM((8, _CHUNK), jnp.int32),
            pltpu.VMEM((_CHUNK,), jnp.float32),
            pltpu.SemaphoreType.DMA,
        ])
    def sc_deg(coli_hbm, z1_hbm, ones_hbm, deg_hbm, deg_s, cidx8, ones_v,
               ssem):
        c = lax.axis_index("c")
        s = lax.axis_index("s")
        r_lo = s * ZT
        pltpu.sync_copy(ones_hbm, ones_v)
        pltpu.sync_copy(z1_hbm, deg_s.at[pl.ds(r_lo, ZT)])
        plsc.subcore_barrier()
        for half in range(_NC):
            def deg_body(jb, _):
                pltpu.sync_copy(coli_hbm.at[half, s, pl.ds(jb * 8, 8)],
                                cidx8)
                for jj in range(8):
                    pltpu.async_copy(ones_v, deg_s.at[cidx8.at[jj]], ssem,
                                     add=True).wait()
                return 0
            lax.fori_loop(0, NCH // 8, deg_body, 0)
        plsc.subcore_barrier()

        @pl.when(c == 0)
        def _():
            pltpu.sync_copy(deg_s.at[pl.ds(r_lo, ZT)],
                            deg_hbm.at[pl.ds(r_lo, ZT)])

    return sc_deg


def _make_sc_hop(NP, D, NCH):
    ZT = NP // _NS
    ZC = 160
    assert ZT % ZC == 0

    @functools.partial(
        pl.kernel,
        out_type=jax.ShapeDtypeStruct((_NC, NP, D), jnp.float32),
        mesh=_mesh(), compiler_params=_CP,
        scratch_types=[
            pltpu.VMEM_SHARED((NP, D), jnp.float32),
            pltpu.VMEM((2, 8, _CHUNK), jnp.int32),
            pltpu.VMEM((2, 8, _CHUNK), jnp.int32),
            pltpu.VMEM((2, _CHUNK, D), jnp.float32),
            pltpu.SemaphoreType.DMA,
            pltpu.SemaphoreType.DMA,
            pltpu.SemaphoreType.DMA,
        ])
    def sc_hop(u_hbm, rowi_hbm, coli_hbm, z2_hbm, t_hbm,
               acc_s, ridx, cidx, gbuf, gsem, ssem, isem):
        c = lax.axis_index("c")
        s = lax.axis_index("s")
        r_lo = s * ZT
        for q in range(ZT // ZC):
            pltpu.sync_copy(z2_hbm, acc_s.at[pl.ds(r_lo + q * ZC, ZC)])
        plsc.subcore_barrier()

        NB = NCH // 8
        T = NB * 8

        def r_at(q):
            return ridx.at[(q // 8) % 2, q % 8]

        def c_at(q):
            return cidx.at[(q // 8) % 2, q % 8]

        pltpu.sync_copy(rowi_hbm.at[c, s, pl.ds(0, 8)], ridx.at[0])
        pltpu.sync_copy(coli_hbm.at[c, s, pl.ds(0, 8)], cidx.at[0])
        g = pltpu.async_copy(u_hbm.at[r_at(0)], gbuf.at[0], gsem)
        ifr = ifc = None
        scats = [None, None]
        for q in range(T):
            jb = q // 8
            if q % 8 == 0 and jb + 1 < NB:
                ifr = pltpu.async_copy(
                    rowi_hbm.at[c, s, pl.ds((jb + 1) * 8, 8)],
                    ridx.at[(jb + 1) % 2], isem)
                ifc = pltpu.async_copy(
                    coli_hbm.at[c, s, pl.ds((jb + 1) * 8, 8)],
                    cidx.at[(jb + 1) % 2], isem)
            g.wait()
            sd = pltpu.async_copy(gbuf.at[q % 2], acc_s.at[c_at(q)], ssem,
                                  add=True)
            if q + 1 < T:
                if (q + 1) % 8 == 0:
                    ifr.wait()
                    ifc.wait()
                if scats[(q + 1) % 2] is not None:
                    scats[(q + 1) % 2].wait()
                    scats[(q + 1) % 2] = None
                g = pltpu.async_copy(u_hbm.at[r_at(q + 1)],
                                     gbuf.at[(q + 1) % 2], gsem)
            scats[q % 2] = sd
        for sd in scats:
            if sd is not None:
                sd.wait()
        plsc.subcore_barrier()

        pltpu.sync_copy(acc_s.at[pl.ds(r_lo, ZT)],
                        t_hbm.at[c, pl.ds(r_lo, ZT)])

    return sc_hop


def _branch_body(N, is_hop, has_acc, emit_u):
    def body(*refs):
        it = iter(refs)
        xin_ref = next(it)
        deg_ref = next(it)
        Wk_ref, bk_ref, gk_ref, bek_ref, Wok_ref = (
            next(it), next(it), next(it), next(it), next(it))
        bo_ref = None if has_acc else next(it)
        acc_ref = next(it) if has_acc else None
        o_ref = next(it)
        u_ref = next(it) if emit_u else None

        deg = deg_ref[...]
        dis = jnp.where(deg > 0.5, lax.rsqrt(deg), 0.0)
        if is_hop:
            xk = (xin_ref[0, :N] + xin_ref[1, :N]) * dis
        else:
            xk = xin_ref[...]
        if emit_u:
            u_ref[...] = xk * dis
        h = lax.dot_general(
            xk, Wk_ref[...], (((1,), (1,)), ((), ())),
            preferred_element_type=jnp.float32,
            precision=lax.Precision.DEFAULT) + bk_ref[...][None, :]
        mean = jnp.mean(h, axis=0, keepdims=True)
        var = jnp.mean(jnp.square(h - mean), axis=0, keepdims=True)
        h = (h - mean) * lax.rsqrt(var + _BN_EPS) * gk_ref[...][None, :]
        h = h + bek_ref[...][None, :]
        h = jnp.maximum(h, 0.0)
        p = lax.dot_general(
            h, Wok_ref[...], (((1,), (1,)), ((), ())),
            preferred_element_type=jnp.float32,
            precision=lax.Precision.DEFAULT)
        if has_acc:
            o_ref[...] = acc_ref[...] + p
        else:
            o_ref[...] = p + bo_ref[...][None, :]

    return body


def kernel(x, W, b, gamma, beta, W_out, b_out, edge_index, num_nodes):
    N, D = x.shape
    K = W.shape[0] - 1
    E = edge_index.shape[1]
    NP = _cdiv(N + 16, 512) * 512
    NW = _NC * _NS

    row = edge_index[0]
    col = edge_index[1]

    assert E % NW == 0
    ET = E // NW
    ETP = _cdiv(ET, _CHUNK * 8) * (_CHUNK * 8)
    NCH = ETP // _CHUNK
    npad = ETP - ET
    lanes = (jnp.arange(npad, dtype=jnp.int32) % 16)
    row_t = jnp.concatenate(
        [row.reshape(NW, ET),
         jnp.broadcast_to(lanes[None, :], (NW, npad))], axis=1)
    col_t = jnp.concatenate(
        [col.reshape(NW, ET),
         jnp.broadcast_to(N + lanes[None, :], (NW, npad))], axis=1)
    row_t = row_t.reshape(_NC, _NS, NCH, _CHUNK)
    col_t = col_t.reshape(_NC, _NS, NCH, _CHUNK)

    ZT = NP // _NS
    z1 = jnp.zeros((ZT,), jnp.float32)
    z2 = jnp.zeros((160, D), jnp.float32)
    ones = jnp.ones((_CHUNK,), jnp.float32)

    sc_deg = _make_sc_deg(NP, NCH)
    deg = sc_deg(col_t, z1, ones)
    deg_n = deg[:N, None]

    sc_hop = _make_sc_hop(NP, D, NCH)
    D_OUT = W_out.shape[0]
    D_HID = W.shape[1]
    acc = None
    xin = x
    for k in range(K + 1):
        emit_u = k < K
        out_shape = [jax.ShapeDtypeStruct((N, D_OUT), jnp.float32)]
        if emit_u:
            out_shape.append(jax.ShapeDtypeStruct((N, D), jnp.float32))
        args = [xin, deg_n, W[k], b[k], gamma[k], beta[k],
                W_out[:, k * D_HID:(k + 1) * D_HID]]
        args.append(b_out if acc is None else acc)
        res = pl.pallas_call(
            _branch_body(N, k > 0, acc is not None, emit_u),
            out_shape=out_shape,
        )(*args)
        acc = res[0]
        if emit_u:
            xin = sc_hop(res[1], row_t, col_t, z2)
    return acc


# speedup vs baseline: 1.0222x; 1.0078x over previous
import functools

import jax
import jax.numpy as jnp
from jax import lax
from jax.experimental import pallas as pl
from jax.experimental.pallas import tpu as pltpu
from jax.experimental.pallas import tpu_sc as plsc

_NC = 2
_NS = 16
_CHUNK = 128
_BN_EPS = 1e-5

_CP = pltpu.CompilerParams(needs_layout_passes=False)


def _mesh():
    return plsc.VectorSubcoreMesh(core_axis_name="c", subcore_axis_name="s",
                                  num_cores=_NC, num_subcores=_NS)


def _cdiv(a, b):
    return (a + b - 1) // b


def _make_sc_deg(NP, NCH):
    ZT = NP // _NS

    @functools.partial(
        pl.kernel, out_type=jax.ShapeDtypeStruct((NP,), jnp.float32),
        mesh=_mesh(), compiler_params=_CP,
        scratch_types=[
            pltpu.VMEM_SHARED((NP,), jnp.float32),
            pltpu.VMEM((8, _CHUNK), jnp.int32),
            pltpu.VMEM((_CHUNK,), jnp.float32),
            pltpu.SemaphoreType.DMA,
        ])
    def sc_deg(coli_hbm, z1_hbm, ones_hbm, deg_hbm, deg_s, cidx8, ones_v,
               ssem):
        c = lax.axis_index("c")
        s = lax.axis_index("s")
        r_lo = s * ZT
        pltpu.sync_copy(ones_hbm, ones_v)
        pltpu.sync_copy(z1_hbm, deg_s.at[pl.ds(r_lo, ZT)])
        plsc.subcore_barrier()
        for half in range(_NC):
            def deg_body(jb, _):
                pltpu.sync_copy(coli_hbm.at[half, s, pl.ds(jb * 8, 8)],
                                cidx8)
                for jj in range(8):
                    pltpu.async_copy(ones_v, deg_s.at[cidx8.at[jj]], ssem,
                                     add=True).wait()
                return 0
            lax.fori_loop(0, NCH // 8, deg_body, 0)
        plsc.subcore_barrier()

        @pl.when(c == 0)
        def _():
            pltpu.sync_copy(deg_s.at[pl.ds(r_lo, ZT)],
                            deg_hbm.at[pl.ds(r_lo, ZT)])

    return sc_deg


def _make_sc_hop(NP, D, NCH):
    ZT = NP // _NS
    ZC = 160
    assert ZT % ZC == 0

    @functools.partial(
        pl.kernel,
        out_type=jax.ShapeDtypeStruct((_NC, NP, D), jnp.float32),
        mesh=_mesh(), compiler_params=_CP,
        scratch_types=[
            pltpu.VMEM_SHARED((NP, D), jnp.float32),
            pltpu.VMEM((2, 8, _CHUNK), jnp.int32),
            pltpu.VMEM((2, 8, _CHUNK), jnp.int32),
            pltpu.VMEM((2, _CHUNK, D), jnp.float32),
            pltpu.SemaphoreType.DMA,
            pltpu.SemaphoreType.DMA,
            pltpu.SemaphoreType.DMA,
        ])
    def sc_hop(u_hbm, rowi_hbm, coli_hbm, z2_hbm, t_hbm,
               acc_s, ridx, cidx, gbuf, gsem, ssem, isem):
        c = lax.axis_index("c")
        s = lax.axis_index("s")
        r_lo = s * ZT

        NB = NCH // 8
        T = NB * 8

        def r_at(q):
            return ridx.at[(q // 8) % 2, q % 8]

        def c_at(q):
            return cidx.at[(q // 8) % 2, q % 8]

        zds = [pltpu.async_copy(z2_hbm, acc_s.at[pl.ds(r_lo + q * ZC, ZC)],
                                isem)
               for q in range(ZT // ZC)]
        pltpu.sync_copy(rowi_hbm.at[c, s, pl.ds(0, 8)], ridx.at[0])
        pltpu.sync_copy(coli_hbm.at[c, s, pl.ds(0, 8)], cidx.at[0])
        g = pltpu.async_copy(u_hbm.at[r_at(0)], gbuf.at[0], gsem)
        for zd in zds:
            zd.wait()
        plsc.subcore_barrier()
        ifr = ifc = None
        scats = [None, None]
        for q in range(T):
            jb = q // 8
            if q % 8 == 0 and jb + 1 < NB:
                ifr = pltpu.async_copy(
                    rowi_hbm.at[c, s, pl.ds((jb + 1) * 8, 8)],
                    ridx.at[(jb + 1) % 2], isem)
                ifc = pltpu.async_copy(
                    coli_hbm.at[c, s, pl.ds((jb + 1) * 8, 8)],
                    cidx.at[(jb + 1) % 2], isem)
            g.wait()
            sd = pltpu.async_copy(gbuf.at[q % 2], acc_s.at[c_at(q)], ssem,
                                  add=True)
            if q + 1 < T:
                if (q + 1) % 8 == 0:
                    ifr.wait()
                    ifc.wait()
                if scats[(q + 1) % 2] is not None:
                    scats[(q + 1) % 2].wait()
                    scats[(q + 1) % 2] = None
                g = pltpu.async_copy(u_hbm.at[r_at(q + 1)],
                                     gbuf.at[(q + 1) % 2], gsem)
            scats[q % 2] = sd
        for sd in scats:
            if sd is not None:
                sd.wait()
        plsc.subcore_barrier()

        pltpu.sync_copy(acc_s.at[pl.ds(r_lo, ZT)],
                        t_hbm.at[c, pl.ds(r_lo, ZT)])

    return sc_hop


def _branch_body(N, is_hop, has_acc, emit_u):
    def body(*refs):
        it = iter(refs)
        xin_ref = next(it)
        deg_ref = next(it)
        Wk_ref, bk_ref, gk_ref, bek_ref, Wok_ref = (
            next(it), next(it), next(it), next(it), next(it))
        bo_ref = None if has_acc else next(it)
        acc_ref = next(it) if has_acc else None
        o_ref = next(it)
        u_ref = next(it) if emit_u else None

        deg = deg_ref[...]
        dis = jnp.where(deg > 0.5, lax.rsqrt(deg), 0.0)
        if is_hop:
            xk = (xin_ref[0, :N] + xin_ref[1, :N]) * dis
        else:
            xk = xin_ref[...]
        if emit_u:
            u_ref[...] = xk * dis
        h = lax.dot_general(
            xk, Wk_ref[...], (((1,), (1,)), ((), ())),
            preferred_element_type=jnp.float32,
            precision=lax.Precision.DEFAULT) + bk_ref[...][None, :]
        mean = jnp.mean(h, axis=0, keepdims=True)
        var = jnp.mean(jnp.square(h - mean), axis=0, keepdims=True)
        h = (h - mean) * lax.rsqrt(var + _BN_EPS) * gk_ref[...][None, :]
        h = h + bek_ref[...][None, :]
        h = jnp.maximum(h, 0.0)
        p = lax.dot_general(
            h, Wok_ref[...], (((1,), (1,)), ((), ())),
            preferred_element_type=jnp.float32,
            precision=lax.Precision.DEFAULT)
        if has_acc:
            o_ref[...] = acc_ref[...] + p
        else:
            o_ref[...] = p + bo_ref[...][None, :]

    return body


def kernel(x, W, b, gamma, beta, W_out, b_out, edge_index, num_nodes):
    N, D = x.shape
    K = W.shape[0] - 1
    E = edge_index.shape[1]
    NP = _cdiv(N + 16, 512) * 512
    NW = _NC * _NS

    row = edge_index[0]
    col = edge_index[1]

    assert E % NW == 0
    ET = E // NW
    ETP = _cdiv(ET, _CHUNK * 8) * (_CHUNK * 8)
    NCH = ETP // _CHUNK
    npad = ETP - ET
    lanes = (jnp.arange(npad, dtype=jnp.int32) % 16)
    row_t = jnp.concatenate(
        [row.reshape(NW, ET),
         jnp.broadcast_to(lanes[None, :], (NW, npad))], axis=1)
    col_t = jnp.concatenate(
        [col.reshape(NW, ET),
         jnp.broadcast_to(N + lanes[None, :], (NW, npad))], axis=1)
    row_t = row_t.reshape(_NC, _NS, NCH, _CHUNK)
    col_t = col_t.reshape(_NC, _NS, NCH, _CHUNK)

    ZT = NP // _NS
    z1 = jnp.zeros((ZT,), jnp.float32)
    z2 = jnp.zeros((160, D), jnp.float32)
    ones = jnp.ones((_CHUNK,), jnp.float32)

    sc_deg = _make_sc_deg(NP, NCH)
    deg = sc_deg(col_t, z1, ones)
    deg_n = deg[:N, None]

    sc_hop = _make_sc_hop(NP, D, NCH)
    D_OUT = W_out.shape[0]
    D_HID = W.shape[1]
    acc = None
    xin = x
    for k in range(K + 1):
        emit_u = k < K
        out_shape = [jax.ShapeDtypeStruct((N, D_OUT), jnp.float32)]
        if emit_u:
            out_shape.append(jax.ShapeDtypeStruct((N, D), jnp.float32))
        args = [xin, deg_n, W[k], b[k], gamma[k], beta[k],
                W_out[:, k * D_HID:(k + 1) * D_HID]]
        args.append(b_out if acc is None else acc)
        res = pl.pallas_call(
            _branch_body(N, k > 0, acc is not None, emit_u),
            out_shape=out_shape,
        )(*args)
        acc = res[0]
        if emit_u:
            xin = sc_hop(res[1], row_t, col_t, z2)
    return acc
